# Initial kernel scaffold; baseline (speedup 1.0000x reference)
import jax
import jax.numpy as jnp
from jax.experimental import pallas as pl


def kernel(user_idx, item_idx, neg_item_idx, feat_user, feat_item, su0, du0, wu0, alu0, aru0, su1, du1, wu1, alu1, aru1, si0, di0, wi0, ali0, ari0, si1, di1, wi1, ali1, ari1, swa_u, sba_u, swb_u, swa_i, sba_i, swb_i, ulw, ulb, ilw, ilb, lng, lnb):
    raise NotImplementedError("write your pallas kernel here")


# speedup vs baseline: 42.5980x; 42.5980x over previous
import dataclasses
import functools

import jax
import jax.numpy as jnp
from jax import lax
from jax.experimental import pallas as pl
from jax.experimental.pallas import tpu as pltpu
from jax.experimental.pallas import tpu_sc as plsc

N = 10000
E = 320000
D = 128
DH = 64
NC = 2
NS = 16
L = 16
NW = NC * NS
EPT = E // NW
CH = 80
NCHUNK = EPT // CH
N2 = 10240
RPT = N2 // NS
ZR = 128
BLK = 1024
DR = N2 // D


def _sc_compiler_params():
    cp = pltpu.CompilerParams()
    fields = getattr(pltpu.CompilerParams, "__dataclass_fields__", {})
    if "needs_layout_passes" in fields:
        cp = dataclasses.replace(cp, needs_layout_passes=False)
    if "use_tc_tiling_on_sc" in fields:
        cp = dataclasses.replace(cp, use_tc_tiling_on_sc=False)
    return cp


def _tc_prep(x, w2, al2, ar2):
    def body(x_ref, w_ref, al_ref, ar_ref, fa_ref, fb_ref, el_ref, er_ref):
        i = pl.program_id(1)
        feat = jnp.dot(x_ref[...], w_ref[0],
                       preferred_element_type=jnp.float32,
                       precision=lax.Precision.HIGHEST)
        fa_ref[0] = feat[:, :DH]
        fb_ref[0] = feat[:, DH:]
        el_ref[0, 0, pl.ds(i * BLK, BLK)] = jnp.sum(
            feat * al_ref[0, 0][None, :], axis=1)
        er_ref[0, 0, pl.ds(i * BLK, BLK)] = jnp.sum(
            feat * ar_ref[0, 0][None, :], axis=1)

    return pl.pallas_call(
        body,
        grid=(2, N2 // BLK),
        in_specs=[
            pl.BlockSpec((BLK, D), lambda g, i: (i, 0)),
            pl.BlockSpec((1, D, D), lambda g, i: (g, 0, 0)),
            pl.BlockSpec((1, 1, D), lambda g, i: (g, 0, 0)),
            pl.BlockSpec((1, 1, D), lambda g, i: (g, 0, 0)),
        ],
        out_specs=[
            pl.BlockSpec((1, BLK, DH), lambda g, i: (g, i, 0)),
            pl.BlockSpec((1, BLK, DH), lambda g, i: (g, i, 0)),
            pl.BlockSpec((1, 1, N2), lambda g, i: (g, 0, 0)),
            pl.BlockSpec((1, 1, N2), lambda g, i: (g, 0, 0)),
        ],
        out_shape=[jax.ShapeDtypeStruct((2, N2, DH), jnp.float32),
                   jax.ShapeDtypeStruct((2, N2, DH), jnp.float32),
                   jax.ShapeDtypeStruct((2, 1, N2), jnp.float32),
                   jax.ShapeDtypeStruct((2, 1, N2), jnp.float32)],
    )(x, w2, al2, ar2)


def _sc_edges(fa, fb, el_flat, er_flat, src0, src1, dst0, dst1):
    mesh = plsc.VectorSubcoreMesh(core_axis_name="c", subcore_axis_name="s")

    @functools.partial(
        pl.kernel,
        out_type=[jax.ShapeDtypeStruct((2 * 2 * NC * N2, DH), jnp.float32),
                  jax.ShapeDtypeStruct((2 * NC * DR, D), jnp.float32)],
        mesh=mesh,
        scratch_types=[
            pltpu.VMEM((N,), jnp.float32),
            pltpu.VMEM((N,), jnp.float32),
            pltpu.VMEM((EPT,), jnp.int32),
            pltpu.VMEM((NCHUNK, CH), jnp.int32),
            pltpu.VMEM((EPT,), jnp.float32),
            pltpu.VMEM((DR, D), jnp.float32),
            pltpu.VMEM((DR,), jnp.int32),
            pltpu.VMEM((CH, DH), jnp.float32),
            pltpu.VMEM((CH, DH), jnp.float32),
            pltpu.VMEM((CH, DH), jnp.float32),
            pltpu.VMEM((CH, DH), jnp.float32),
            pltpu.VMEM((ZR, DH), jnp.float32),
            pltpu.VMEM_SHARED((N2, DH), jnp.float32),
            pltpu.VMEM_SHARED((DR, D), jnp.float32),
            pltpu.SemaphoreType.DMA,
            pltpu.SemaphoreType.DMA,
            pltpu.SemaphoreType.DMA,
            pltpu.SemaphoreType.DMA,
        ],
        compiler_params=_sc_compiler_params(),
    )
    def body(fa_h, fb_h, el_h, er_h, src0_h, src1_h, dst0_h, dst1_h,
             out_h, den_h,
             el_v, er_v, srcb, dstb, sbuf, den_v, rowid,
             grows0, grows1, stage0, stage1,
             zbuf, acc, accd, gsem0, gsem1, ssem0, ssem1):
        c = lax.axis_index("c")
        s = lax.axis_index("s")
        wid = c * NS + s
        zvec = jnp.zeros((L,), jnp.float32)
        lane = lax.iota(jnp.int32, L)
        lane_masks = [lane == j for j in range(L)]

        @pl.loop(0, ZR)
        def _(r):
            for q in range(DH // L):
                zbuf[r, pl.ds(q * L, L)] = zvec

        for k in range(DR // L):
            rowid[pl.ds(k * L, L)] = lane + k * L

        src_refs = (src0_h, src1_h)
        dst_refs = (dst0_h, dst1_h)
        for g in range(2):
            pltpu.sync_copy(el_h.at[pl.ds(g * N2, N)], el_v)
            pltpu.sync_copy(er_h.at[pl.ds(g * N2, N)], er_v)
            pltpu.sync_copy(src_refs[g].at[pl.ds(wid * EPT, EPT)], srcb)
            pltpu.sync_copy(dst_refs[g].at[pl.ds(wid * NCHUNK, NCHUNK)], dstb)

            @pl.loop(0, DR)
            def _(k):
                for q in range(D // L):
                    den_v[k, pl.ds(q * L, L)] = zvec

            @pl.when(s == 0)
            def _():
                pltpu.sync_copy(den_v, accd)

            @pl.loop(0, NCHUNK)
            def _(r):
                for u in range(CH // L):
                    sl = pl.ds(r * CH + u * L, L)
                    dv = dstb[r, pl.ds(u * L, L)]
                    ev = (plsc.load_gather(el_v, [srcb[sl]])
                          + plsc.load_gather(er_v, [dv]))
                    ev = jnp.where(ev > 0.0, ev, ev * 0.2)
                    sval = jnp.exp(ev)
                    sbuf[sl] = sval
                    drow = lax.shift_right_logical(dv, 7)
                    dcol = jnp.bitwise_and(dv, 127)
                    for j in range(L):
                        plsc.addupdate_scatter(den_v, [drow, dcol], sval,
                                               mask=lane_masks[j])

            def weight(k, grows, stage):
                @plsc.parallel_loop(0, CH // L)
                def _(v):
                    sv = sbuf[pl.ds(k * CH + v * L, L)]
                    for j in range(L):
                        e = v * L + j
                        spl = jnp.broadcast_to(sv[j], (L,))
                        for q in range(DH // L):
                            cs = pl.ds(q * L, L)
                            stage[e, cs] = grows[e, cs] * spl

            for half in range(2):
                tab_g = (fa_h if half == 0 else fb_h).at[pl.ds(g * N2, N2)]
                for z in range(RPT // ZR):
                    pltpu.sync_copy(zbuf, acc.at[pl.ds(s * RPT + z * ZR, ZR)])
                plsc.subcore_barrier()
                if half == 0:
                    pltpu.sync_copy(den_v, accd.at[rowid], add=True)

                def gather(k, grows, gsem):
                    idx = srcb.at[pl.ds(k * CH, CH)]
                    return pltpu.async_copy(tab_g.at[idx], grows, gsem)

                def gather_wait(grows, gsem):
                    idx = srcb.at[pl.ds(0, CH)]
                    pltpu.make_async_copy(tab_g.at[idx], grows, gsem).wait()

                def scatter(k, stage, ssem):
                    return pltpu.async_copy(stage, acc.at[dstb.at[k]], ssem,
                                            add=True)

                def scatter_wait(stage, ssem):
                    pltpu.make_async_copy(stage, acc.at[dstb.at[0]],
                                          ssem).wait()

                NPAIR = (NCHUNK + 1) // 2
                gather(0, grows0, gsem0)
                gather(1, grows1, gsem1)

                @pl.loop(0, NPAIR)
                def _(i):
                    k0 = 2 * i
                    k1 = k0 + 1

                    @pl.when(i > 0)
                    def _():
                        scatter_wait(stage0, ssem0)
                        scatter_wait(stage1, ssem1)

                    gather_wait(grows0, gsem0)
                    weight(k0, grows0, stage0)
                    scatter(k0, stage0, ssem0)

                    @pl.when(i < NPAIR - 1)
                    def _():
                        gather(k0 + 2, grows0, gsem0)

                    @pl.when(i < NPAIR - 1)
                    def _():
                        gather_wait(grows1, gsem1)
                        weight(k1, grows1, stage1)
                        scatter(k1, stage1, ssem1)

                        @pl.when(i < NPAIR - 2)
                        def _():
                            gather(k1 + 2, grows1, gsem1)

                scatter_wait(stage0, ssem0)

                plsc.subcore_barrier()
                if half == 0:
                    @pl.when(s == 0)
                    def _():
                        pltpu.sync_copy(
                            accd, den_h.at[pl.ds((g * NC + c) * DR, DR)])
                obase = ((g * 2 + half) * NC) * N2
                for z in range(RPT // ZR):
                    rs = s * RPT + z * ZR
                    pltpu.sync_copy(acc.at[pl.ds(rs, ZR)],
                                    out_h.at[pl.ds(obase + c * N2 + rs, ZR)])
                plsc.subcore_barrier()

    return body(fa, fb, el_flat, er_flat, src0, src1, dst0, dst1)


def _tc_combine(partials, dens, swa, sba, swb):
    def body(pa_ref, pb_ref, d_ref, swa_ref, sba_ref, swb_ref, h_ref, pr_ref):
        i = pl.program_id(1)
        pa = pa_ref[0, 0]
        pb = pb_ref[0, 0]
        numer = jnp.concatenate([pa[0] + pa[1], pb[0] + pb[1]], axis=-1)
        den = d_ref[0, 0] + d_ref[0, 1]
        x = numer / (den[:, None] + 1e-9)
        h = jnp.where(x > 0.0, x, jnp.exp(x) - 1.0)
        h_ref[0] = h
        t = jnp.tanh(jnp.dot(h, swa_ref[...],
                             preferred_element_type=jnp.float32,
                             precision=lax.Precision.HIGHEST)
                     + sba_ref[0][None, :])
        pr = jnp.sum(t * swb_ref[0][None, :], axis=1)
        row = i * BLK + lax.broadcasted_iota(jnp.int32, (BLK,), 0)
        pr = jnp.where(row < N, pr, 0.0)
        pr_ref[0, 0, pl.ds(i * BLK, BLK)] = pr

    return pl.pallas_call(
        body,
        grid=(2, N2 // BLK),
        in_specs=[
            pl.BlockSpec((1, 1, NC, BLK, DH), lambda g, i: (g, 0, 0, i, 0)),
            pl.BlockSpec((1, 1, NC, BLK, DH), lambda g, i: (g, 1, 0, i, 0)),
            pl.BlockSpec((1, NC, BLK), lambda g, i: (g, 0, i)),
            pl.BlockSpec((D, D), lambda g, i: (0, 0)),
            pl.BlockSpec((1, D), lambda g, i: (0, 0)),
            pl.BlockSpec((1, D), lambda g, i: (0, 0)),
        ],
        out_specs=[
            pl.BlockSpec((1, BLK, D), lambda g, i: (g, i, 0)),
            pl.BlockSpec((1, 1, N2), lambda g, i: (g, 0, 0)),
        ],
        out_shape=[jax.ShapeDtypeStruct((2, N2, D), jnp.float32),
                   jax.ShapeDtypeStruct((2, 1, N2), jnp.float32)],
    )(partials, partials, dens, swa, sba, swb)


def _tc_epilogue(h, p, lw, lb, lng1, lnb1):
    def body(ha_ref, hb_ref, p_ref, lw_ref, lb_ref, lng_ref, lnb_ref, o_ref):
        w = jnp.sum(p_ref[...], axis=(1, 2)) / N
        m = jnp.maximum(w[0], w[1])
        e0 = jnp.exp(w[0] - m)
        e1 = jnp.exp(w[1] - m)
        b0 = e0 / (e0 + e1)
        b1 = e1 / (e0 + e1)
        x = b0 * ha_ref[0] + b1 * hb_ref[0]
        y = jnp.dot(x, lw_ref[...], preferred_element_type=jnp.float32,
                    precision=lax.Precision.HIGHEST) + lb_ref[0][None, :]
        y = jnp.maximum(y, 0.0)
        mu = jnp.mean(y, axis=1, keepdims=True)
        var = jnp.mean((y - mu) ** 2, axis=1, keepdims=True)
        o_ref[...] = ((y - mu) / jnp.sqrt(var + 1e-5)
                      * lng_ref[0][None, :] + lnb_ref[0][None, :])

    return pl.pallas_call(
        body,
        grid=(N2 // BLK,),
        in_specs=[
            pl.BlockSpec((1, BLK, D), lambda i: (0, i, 0)),
            pl.BlockSpec((1, BLK, D), lambda i: (1, i, 0)),
            pl.BlockSpec((2, 1, N2), lambda i: (0, 0, 0)),
            pl.BlockSpec((D, D), lambda i: (0, 0)),
            pl.BlockSpec((1, D), lambda i: (0, 0)),
            pl.BlockSpec((1, D), lambda i: (0, 0)),
            pl.BlockSpec((1, D), lambda i: (0, 0)),
        ],
        out_specs=pl.BlockSpec((BLK, D), lambda i: (i, 0)),
        out_shape=jax.ShapeDtypeStruct((N2, D), jnp.float32),
    )(h, h, p, lw, lb, lng1, lnb1)


def _sc_gather(tab_u, tab_i, idx_u, idx_i):
    mesh = plsc.VectorSubcoreMesh(core_axis_name="c", subcore_axis_name="s")
    BU = idx_u.shape[0]
    BI = idx_i.shape[0]
    upt = BU // NW
    ipt = BI // NW
    GCH = 128

    @functools.partial(
        pl.kernel,
        out_type=jax.ShapeDtypeStruct((BU + BI, D), jnp.float32),
        mesh=mesh,
        scratch_types=[
            pltpu.VMEM((GCH,), jnp.int32),
            pltpu.VMEM((GCH, D), jnp.float32),
            pltpu.SemaphoreType.DMA,
        ],
        compiler_params=_sc_compiler_params(),
    )
    def body(tu_h, ti_h, iu_h, ii_h, out_h, idxb, rows, sem):
        c = lax.axis_index("c")
        s = lax.axis_index("s")
        wid = c * NS + s
        for ck in range(upt // GCH):
            base = wid * upt + ck * GCH
            pltpu.sync_copy(iu_h.at[pl.ds(base, GCH)], idxb)
            pltpu.async_copy(tu_h.at[idxb], rows, sem).wait()
            pltpu.sync_copy(rows, out_h.at[pl.ds(base, GCH)])
        for ck in range(ipt // GCH):
            base = wid * ipt + ck * GCH
            pltpu.sync_copy(ii_h.at[pl.ds(base, GCH)], idxb)
            pltpu.async_copy(ti_h.at[idxb], rows, sem).wait()
            pltpu.sync_copy(rows, out_h.at[pl.ds(BU + base, GCH)])

    return body(tab_u, tab_i, idx_u, idx_i)


def _run_type(x, w0, w1, al0, al1, ar0, ar1, swa, sba, swb, lw, lb,
              lng1, lnb1, src0, src1, dst0, dst1):
    f32 = jnp.float32
    w2 = jnp.stack([w0, w1]).astype(f32)
    al2 = jnp.stack([al0.reshape(-1), al1.reshape(-1)]).astype(f32)
    al2 = al2.reshape(2, 1, D)
    ar2 = jnp.stack([ar0.reshape(-1), ar1.reshape(-1)]).astype(f32)
    ar2 = ar2.reshape(2, 1, D)
    fa, fb, el, er = _tc_prep(x.astype(f32), w2, al2, ar2)
    partials_flat, dens_flat = _sc_edges(
        fa.reshape(2 * N2, DH), fb.reshape(2 * N2, DH),
        el.reshape(2 * N2), er.reshape(2 * N2),
        src0.astype(jnp.int32), src1.astype(jnp.int32),
        dst0.astype(jnp.int32).reshape(NW * NCHUNK, CH),
        dst1.astype(jnp.int32).reshape(NW * NCHUNK, CH))
    partials = partials_flat.reshape(2, 2, NC, N2, DH)
    dens = dens_flat.reshape(2, NC, N2)
    h, p = _tc_combine(partials, dens, swa.astype(f32),
                       sba.astype(f32).reshape(1, D),
                       swb.astype(f32).reshape(1, D))
    return _tc_epilogue(h, p, lw.astype(f32), lb.astype(f32).reshape(1, D),
                        lng1, lnb1)


def kernel(user_idx, item_idx, neg_item_idx, feat_user, feat_item,
           su0, du0, wu0, alu0, aru0, su1, du1, wu1, alu1, aru1,
           si0, di0, wi0, ali0, ari0, si1, di1, wi1, ali1, ari1,
           swa_u, sba_u, swb_u, swa_i, sba_i, swb_i,
           ulw, ulb, ilw, ilb, lng, lnb):
    f32 = jnp.float32
    lng1 = lng.reshape(1, D).astype(f32)
    lnb1 = lnb.reshape(1, D).astype(f32)

    emb_u = _run_type(feat_user, wu0, wu1, alu0, alu1, aru0, aru1,
                      swa_u, sba_u, swb_u, ulw, ulb, lng1, lnb1,
                      su0, su1, du0, du1)
    emb_i = _run_type(feat_item, wi0, wi1, ali0, ali1, ari0, ari1,
                      swa_i, sba_i, swb_i, ilw, ilb, lng1, lnb1,
                      si0, si1, di0, di1)

    idx_u = user_idx.astype(jnp.int32)
    idx_i = jnp.concatenate([item_idx, neg_item_idx]).astype(jnp.int32)
    gathered = _sc_gather(emb_u, emb_i, idx_u, idx_i)
    b = user_idx.shape[0]
    return (gathered[:b], gathered[b:2 * b], gathered[2 * b:])


# speedup vs baseline: 43.3637x; 1.0180x over previous
import dataclasses
import functools

import jax
import jax.numpy as jnp
from jax import lax
from jax.experimental import pallas as pl
from jax.experimental.pallas import tpu as pltpu
from jax.experimental.pallas import tpu_sc as plsc

N = 10000
E = 320000
D = 128
DH = 64
NC = 2
NS = 16
L = 16
NW = NC * NS
EPT = E // NW
CH = 80
NCHUNK = EPT // CH
N2 = 10240
RPT = N2 // NS
ZR = 128
BLK = 1024
DR = N2 // D


def _sc_compiler_params():
    cp = pltpu.CompilerParams()
    fields = getattr(pltpu.CompilerParams, "__dataclass_fields__", {})
    if "needs_layout_passes" in fields:
        cp = dataclasses.replace(cp, needs_layout_passes=False)
    if "use_tc_tiling_on_sc" in fields:
        cp = dataclasses.replace(cp, use_tc_tiling_on_sc=False)
    return cp


def _tc_prep(x, w2, al2, ar2):
    def body(x_ref, w_ref, al_ref, ar_ref, fa_ref, fb_ref, el_ref, er_ref):
        i = pl.program_id(1)
        feat = jnp.dot(x_ref[...], w_ref[0],
                       preferred_element_type=jnp.float32,
                       precision=lax.Precision.HIGHEST)
        fa_ref[0] = feat[:, :DH]
        fb_ref[0] = feat[:, DH:]
        el_ref[0, 0, pl.ds(i * BLK, BLK)] = jnp.sum(
            feat * al_ref[0, 0][None, :], axis=1)
        er_ref[0, 0, pl.ds(i * BLK, BLK)] = jnp.sum(
            feat * ar_ref[0, 0][None, :], axis=1)

    return pl.pallas_call(
        body,
        grid=(2, N2 // BLK),
        in_specs=[
            pl.BlockSpec((BLK, D), lambda g, i: (i, 0)),
            pl.BlockSpec((1, D, D), lambda g, i: (g, 0, 0)),
            pl.BlockSpec((1, 1, D), lambda g, i: (g, 0, 0)),
            pl.BlockSpec((1, 1, D), lambda g, i: (g, 0, 0)),
        ],
        out_specs=[
            pl.BlockSpec((1, BLK, DH), lambda g, i: (g, i, 0)),
            pl.BlockSpec((1, BLK, DH), lambda g, i: (g, i, 0)),
            pl.BlockSpec((1, 1, N2), lambda g, i: (g, 0, 0)),
            pl.BlockSpec((1, 1, N2), lambda g, i: (g, 0, 0)),
        ],
        out_shape=[jax.ShapeDtypeStruct((2, N2, DH), jnp.float32),
                   jax.ShapeDtypeStruct((2, N2, DH), jnp.float32),
                   jax.ShapeDtypeStruct((2, 1, N2), jnp.float32),
                   jax.ShapeDtypeStruct((2, 1, N2), jnp.float32)],
    )(x, w2, al2, ar2)


def _sc_edges(fa, fb, el_flat, er_flat, src0, src1, dst0, dst1):
    mesh = plsc.VectorSubcoreMesh(core_axis_name="c", subcore_axis_name="s")

    @functools.partial(
        pl.kernel,
        out_type=[jax.ShapeDtypeStruct((2 * 2 * NC * N2, DH), jnp.float32),
                  jax.ShapeDtypeStruct((2 * NC * DR, D), jnp.float32)],
        mesh=mesh,
        scratch_types=[
            pltpu.VMEM((N,), jnp.float32),
            pltpu.VMEM((N,), jnp.float32),
            pltpu.VMEM((EPT,), jnp.int32),
            pltpu.VMEM((NCHUNK, CH), jnp.int32),
            pltpu.VMEM((EPT,), jnp.float32),
            pltpu.VMEM((DR, D), jnp.float32),
            pltpu.VMEM((DR,), jnp.int32),
            pltpu.VMEM((CH, DH), jnp.float32),
            pltpu.VMEM((CH, DH), jnp.float32),
            pltpu.VMEM((CH, DH), jnp.float32),
            pltpu.VMEM((CH, DH), jnp.float32),
            pltpu.VMEM((ZR, DH), jnp.float32),
            pltpu.VMEM_SHARED((N2, DH), jnp.float32),
            pltpu.VMEM_SHARED((DR, D), jnp.float32),
            pltpu.SemaphoreType.DMA,
            pltpu.SemaphoreType.DMA,
            pltpu.SemaphoreType.DMA,
            pltpu.SemaphoreType.DMA,
            pltpu.SemaphoreType.DMA,
        ],
        compiler_params=_sc_compiler_params(),
    )
    def body(fa_h, fb_h, el_h, er_h, src0_h, src1_h, dst0_h, dst1_h,
             out_h, den_h,
             el_v, er_v, srcb, dstb, sbuf, den_v, rowid,
             grows0, grows1, stage0, stage1,
             zbuf, acc, accd, gsem0, gsem1, ssem0, ssem1, zsem):
        c = lax.axis_index("c")
        s = lax.axis_index("s")
        wid = c * NS + s
        zvec = jnp.zeros((L,), jnp.float32)
        lane = lax.iota(jnp.int32, L)
        lane_masks = [lane == j for j in range(L)]

        @pl.loop(0, ZR)
        def _(r):
            for q in range(DH // L):
                zbuf[r, pl.ds(q * L, L)] = zvec

        for k in range(DR // L):
            rowid[pl.ds(k * L, L)] = lane + k * L

        src_refs = (src0_h, src1_h)
        dst_refs = (dst0_h, dst1_h)
        def fire_zeros():
            for z in range(RPT // ZR):
                pltpu.async_copy(zbuf, acc.at[pl.ds(s * RPT + z * ZR, ZR)],
                                 zsem)

        def drain_zeros():
            for z in range(RPT // ZR):
                pltpu.make_async_copy(zbuf, acc.at[pl.ds(s * RPT, ZR)],
                                      zsem).wait()

        for g in range(2):
            cel = pltpu.async_copy(el_h.at[pl.ds(g * N2, N)], el_v, gsem0)
            cer = pltpu.async_copy(er_h.at[pl.ds(g * N2, N)], er_v, gsem1)
            csrc = pltpu.async_copy(src_refs[g].at[pl.ds(wid * EPT, EPT)],
                                    srcb, ssem0)
            cdst = pltpu.async_copy(
                dst_refs[g].at[pl.ds(wid * NCHUNK, NCHUNK)], dstb, ssem1)
            fire_zeros()

            @pl.loop(0, DR)
            def _(k):
                for q in range(D // L):
                    den_v[k, pl.ds(q * L, L)] = zvec

            @pl.when(s == 0)
            def _():
                pltpu.sync_copy(den_v, accd)

            cel.wait()
            cer.wait()
            csrc.wait()
            cdst.wait()

            @pl.loop(0, NCHUNK)
            def _(r):
                for u in range(CH // L):
                    sl = pl.ds(r * CH + u * L, L)
                    dv = dstb[r, pl.ds(u * L, L)]
                    ev = (plsc.load_gather(el_v, [srcb[sl]])
                          + plsc.load_gather(er_v, [dv]))
                    ev = jnp.where(ev > 0.0, ev, ev * 0.2)
                    sval = jnp.exp(ev)
                    sbuf[sl] = sval
                    drow = lax.shift_right_logical(dv, 7)
                    dcol = jnp.bitwise_and(dv, 127)
                    for j in range(L):
                        plsc.addupdate_scatter(den_v, [drow, dcol], sval,
                                               mask=lane_masks[j])

            def weight(k, grows, stage):
                @plsc.parallel_loop(0, CH // L)
                def _(v):
                    sv = sbuf[pl.ds(k * CH + v * L, L)]
                    for j in range(L):
                        e = v * L + j
                        spl = jnp.broadcast_to(sv[j], (L,))
                        for q in range(DH // L):
                            cs = pl.ds(q * L, L)
                            stage[e, cs] = grows[e, cs] * spl

            for half in range(2):
                tab_g = (fa_h if half == 0 else fb_h).at[pl.ds(g * N2, N2)]
                if half == 1:
                    fire_zeros()
                drain_zeros()
                plsc.subcore_barrier()
                if half == 0:
                    pltpu.sync_copy(den_v, accd.at[rowid], add=True)

                def gather(k, grows, gsem):
                    idx = srcb.at[pl.ds(k * CH, CH)]
                    return pltpu.async_copy(tab_g.at[idx], grows, gsem)

                def gather_wait(grows, gsem):
                    idx = srcb.at[pl.ds(0, CH)]
                    pltpu.make_async_copy(tab_g.at[idx], grows, gsem).wait()

                def scatter(k, stage, ssem):
                    return pltpu.async_copy(stage, acc.at[dstb.at[k]], ssem,
                                            add=True)

                def scatter_wait(stage, ssem):
                    pltpu.make_async_copy(stage, acc.at[dstb.at[0]],
                                          ssem).wait()

                NPAIR = (NCHUNK + 1) // 2
                gather(0, grows0, gsem0)
                gather(1, grows1, gsem1)

                @pl.loop(0, NPAIR)
                def _(i):
                    k0 = 2 * i
                    k1 = k0 + 1

                    @pl.when(i > 0)
                    def _():
                        scatter_wait(stage0, ssem0)
                        scatter_wait(stage1, ssem1)

                    gather_wait(grows0, gsem0)
                    weight(k0, grows0, stage0)
                    scatter(k0, stage0, ssem0)

                    @pl.when(i < NPAIR - 1)
                    def _():
                        gather(k0 + 2, grows0, gsem0)

                    @pl.when(i < NPAIR - 1)
                    def _():
                        gather_wait(grows1, gsem1)
                        weight(k1, grows1, stage1)
                        scatter(k1, stage1, ssem1)

                        @pl.when(i < NPAIR - 2)
                        def _():
                            gather(k1 + 2, grows1, gsem1)

                scatter_wait(stage0, ssem0)

                plsc.subcore_barrier()
                if half == 0:
                    @pl.when(s == 0)
                    def _():
                        pltpu.sync_copy(
                            accd, den_h.at[pl.ds((g * NC + c) * DR, DR)])
                obase = ((g * 2 + half) * NC) * N2
                for z in range(RPT // ZR):
                    rs = s * RPT + z * ZR
                    pltpu.sync_copy(acc.at[pl.ds(rs, ZR)],
                                    out_h.at[pl.ds(obase + c * N2 + rs, ZR)])
                plsc.subcore_barrier()

    return body(fa, fb, el_flat, er_flat, src0, src1, dst0, dst1)


def _tc_combine(partials, dens, swa, sba, swb):
    def body(pa_ref, pb_ref, d_ref, swa_ref, sba_ref, swb_ref, h_ref, pr_ref):
        i = pl.program_id(1)
        pa = pa_ref[0, 0]
        pb = pb_ref[0, 0]
        numer = jnp.concatenate([pa[0] + pa[1], pb[0] + pb[1]], axis=-1)
        den = d_ref[0, 0] + d_ref[0, 1]
        x = numer / (den[:, None] + 1e-9)
        h = jnp.where(x > 0.0, x, jnp.exp(x) - 1.0)
        h_ref[0] = h
        t = jnp.tanh(jnp.dot(h, swa_ref[...],
                             preferred_element_type=jnp.float32,
                             precision=lax.Precision.HIGHEST)
                     + sba_ref[0][None, :])
        pr = jnp.sum(t * swb_ref[0][None, :], axis=1)
        row = i * BLK + lax.broadcasted_iota(jnp.int32, (BLK,), 0)
        pr = jnp.where(row < N, pr, 0.0)
        pr_ref[0, 0, pl.ds(i * BLK, BLK)] = pr

    return pl.pallas_call(
        body,
        grid=(2, N2 // BLK),
        in_specs=[
            pl.BlockSpec((1, 1, NC, BLK, DH), lambda g, i: (g, 0, 0, i, 0)),
            pl.BlockSpec((1, 1, NC, BLK, DH), lambda g, i: (g, 1, 0, i, 0)),
            pl.BlockSpec((1, NC, BLK), lambda g, i: (g, 0, i)),
            pl.BlockSpec((D, D), lambda g, i: (0, 0)),
            pl.BlockSpec((1, D), lambda g, i: (0, 0)),
            pl.BlockSpec((1, D), lambda g, i: (0, 0)),
        ],
        out_specs=[
            pl.BlockSpec((1, BLK, D), lambda g, i: (g, i, 0)),
            pl.BlockSpec((1, 1, N2), lambda g, i: (g, 0, 0)),
        ],
        out_shape=[jax.ShapeDtypeStruct((2, N2, D), jnp.float32),
                   jax.ShapeDtypeStruct((2, 1, N2), jnp.float32)],
    )(partials, partials, dens, swa, sba, swb)


def _tc_epilogue(h, p, lw, lb, lng1, lnb1):
    def body(ha_ref, hb_ref, p_ref, lw_ref, lb_ref, lng_ref, lnb_ref, o_ref):
        w = jnp.sum(p_ref[...], axis=(1, 2)) / N
        m = jnp.maximum(w[0], w[1])
        e0 = jnp.exp(w[0] - m)
        e1 = jnp.exp(w[1] - m)
        b0 = e0 / (e0 + e1)
        b1 = e1 / (e0 + e1)
        x = b0 * ha_ref[0] + b1 * hb_ref[0]
        y = jnp.dot(x, lw_ref[...], preferred_element_type=jnp.float32,
                    precision=lax.Precision.HIGHEST) + lb_ref[0][None, :]
        y = jnp.maximum(y, 0.0)
        mu = jnp.mean(y, axis=1, keepdims=True)
        var = jnp.mean((y - mu) ** 2, axis=1, keepdims=True)
        o_ref[...] = ((y - mu) / jnp.sqrt(var + 1e-5)
                      * lng_ref[0][None, :] + lnb_ref[0][None, :])

    return pl.pallas_call(
        body,
        grid=(N2 // BLK,),
        in_specs=[
            pl.BlockSpec((1, BLK, D), lambda i: (0, i, 0)),
            pl.BlockSpec((1, BLK, D), lambda i: (1, i, 0)),
            pl.BlockSpec((2, 1, N2), lambda i: (0, 0, 0)),
            pl.BlockSpec((D, D), lambda i: (0, 0)),
            pl.BlockSpec((1, D), lambda i: (0, 0)),
            pl.BlockSpec((1, D), lambda i: (0, 0)),
            pl.BlockSpec((1, D), lambda i: (0, 0)),
        ],
        out_specs=pl.BlockSpec((BLK, D), lambda i: (i, 0)),
        out_shape=jax.ShapeDtypeStruct((N2, D), jnp.float32),
    )(h, h, p, lw, lb, lng1, lnb1)


def _sc_gather(tab_u, tab_i, idx_u, idx_i):
    mesh = plsc.VectorSubcoreMesh(core_axis_name="c", subcore_axis_name="s")
    BU = idx_u.shape[0]
    BI = idx_i.shape[0]
    upt = BU // NW
    ipt = BI // NW
    GCH = 128

    @functools.partial(
        pl.kernel,
        out_type=jax.ShapeDtypeStruct((BU + BI, D), jnp.float32),
        mesh=mesh,
        scratch_types=[
            pltpu.VMEM((GCH,), jnp.int32),
            pltpu.VMEM((GCH, D), jnp.float32),
            pltpu.SemaphoreType.DMA,
        ],
        compiler_params=_sc_compiler_params(),
    )
    def body(tu_h, ti_h, iu_h, ii_h, out_h, idxb, rows, sem):
        c = lax.axis_index("c")
        s = lax.axis_index("s")
        wid = c * NS + s
        for ck in range(upt // GCH):
            base = wid * upt + ck * GCH
            pltpu.sync_copy(iu_h.at[pl.ds(base, GCH)], idxb)
            pltpu.async_copy(tu_h.at[idxb], rows, sem).wait()
            pltpu.sync_copy(rows, out_h.at[pl.ds(base, GCH)])
        for ck in range(ipt // GCH):
            base = wid * ipt + ck * GCH
            pltpu.sync_copy(ii_h.at[pl.ds(base, GCH)], idxb)
            pltpu.async_copy(ti_h.at[idxb], rows, sem).wait()
            pltpu.sync_copy(rows, out_h.at[pl.ds(BU + base, GCH)])

    return body(tab_u, tab_i, idx_u, idx_i)


def _run_type(x, w0, w1, al0, al1, ar0, ar1, swa, sba, swb, lw, lb,
              lng1, lnb1, src0, src1, dst0, dst1):
    f32 = jnp.float32
    w2 = jnp.stack([w0, w1]).astype(f32)
    al2 = jnp.stack([al0.reshape(-1), al1.reshape(-1)]).astype(f32)
    al2 = al2.reshape(2, 1, D)
    ar2 = jnp.stack([ar0.reshape(-1), ar1.reshape(-1)]).astype(f32)
    ar2 = ar2.reshape(2, 1, D)
    fa, fb, el, er = _tc_prep(x.astype(f32), w2, al2, ar2)
    partials_flat, dens_flat = _sc_edges(
        fa.reshape(2 * N2, DH), fb.reshape(2 * N2, DH),
        el.reshape(2 * N2), er.reshape(2 * N2),
        src0.astype(jnp.int32), src1.astype(jnp.int32),
        dst0.astype(jnp.int32).reshape(NW * NCHUNK, CH),
        dst1.astype(jnp.int32).reshape(NW * NCHUNK, CH))
    partials = partials_flat.reshape(2, 2, NC, N2, DH)
    dens = dens_flat.reshape(2, NC, N2)
    h, p = _tc_combine(partials, dens, swa.astype(f32),
                       sba.astype(f32).reshape(1, D),
                       swb.astype(f32).reshape(1, D))
    return _tc_epilogue(h, p, lw.astype(f32), lb.astype(f32).reshape(1, D),
                        lng1, lnb1)


def kernel(user_idx, item_idx, neg_item_idx, feat_user, feat_item,
           su0, du0, wu0, alu0, aru0, su1, du1, wu1, alu1, aru1,
           si0, di0, wi0, ali0, ari0, si1, di1, wi1, ali1, ari1,
           swa_u, sba_u, swb_u, swa_i, sba_i, swb_i,
           ulw, ulb, ilw, ilb, lng, lnb):
    f32 = jnp.float32
    lng1 = lng.reshape(1, D).astype(f32)
    lnb1 = lnb.reshape(1, D).astype(f32)

    emb_u = _run_type(feat_user, wu0, wu1, alu0, alu1, aru0, aru1,
                      swa_u, sba_u, swb_u, ulw, ulb, lng1, lnb1,
                      su0, su1, du0, du1)
    emb_i = _run_type(feat_item, wi0, wi1, ali0, ali1, ari0, ari1,
                      swa_i, sba_i, swb_i, ilw, ilb, lng1, lnb1,
                      si0, si1, di0, di1)

    idx_u = user_idx.astype(jnp.int32)
    idx_i = jnp.concatenate([item_idx, neg_item_idx]).astype(jnp.int32)
    gathered = _sc_gather(emb_u, emb_i, idx_u, idx_i)
    b = user_idx.shape[0]
    return (gathered[:b], gathered[b:2 * b], gathered[2 * b:])


# speedup vs baseline: 44.7709x; 1.0325x over previous
import dataclasses
import functools

import jax
import jax.numpy as jnp
from jax import lax
from jax.experimental import pallas as pl
from jax.experimental.pallas import tpu as pltpu
from jax.experimental.pallas import tpu_sc as plsc

N = 10000
E = 320000
D = 128
DH = 64
NC = 2
NS = 16
L = 16
NW = NC * NS
EPT = E // NW
CH = 80
NCHUNK = EPT // CH
N2 = 10240
RPT = N2 // NS
ZR = 128
BLK = 1024
DR = N2 // D


def _sc_compiler_params():
    cp = pltpu.CompilerParams()
    fields = getattr(pltpu.CompilerParams, "__dataclass_fields__", {})
    if "needs_layout_passes" in fields:
        cp = dataclasses.replace(cp, needs_layout_passes=False)
    if "use_tc_tiling_on_sc" in fields:
        cp = dataclasses.replace(cp, use_tc_tiling_on_sc=False)
    return cp


def _tc_prep(x, w2, al2, ar2):
    def body(x_ref, w_ref, al_ref, ar_ref, fa_ref, fb_ref, el_ref, er_ref):
        i = pl.program_id(1)
        feat = jnp.dot(x_ref[...], w_ref[0],
                       preferred_element_type=jnp.float32)
        fa_ref[0] = feat[:, :DH]
        fb_ref[0] = feat[:, DH:]
        el_ref[0, 0, pl.ds(i * BLK, BLK)] = jnp.sum(
            feat * al_ref[0, 0][None, :], axis=1)
        er_ref[0, 0, pl.ds(i * BLK, BLK)] = jnp.sum(
            feat * ar_ref[0, 0][None, :], axis=1)

    return pl.pallas_call(
        body,
        grid=(2, N2 // BLK),
        in_specs=[
            pl.BlockSpec((BLK, D), lambda g, i: (i, 0)),
            pl.BlockSpec((1, D, D), lambda g, i: (g, 0, 0)),
            pl.BlockSpec((1, 1, D), lambda g, i: (g, 0, 0)),
            pl.BlockSpec((1, 1, D), lambda g, i: (g, 0, 0)),
        ],
        out_specs=[
            pl.BlockSpec((1, BLK, DH), lambda g, i: (g, i, 0)),
            pl.BlockSpec((1, BLK, DH), lambda g, i: (g, i, 0)),
            pl.BlockSpec((1, 1, N2), lambda g, i: (g, 0, 0)),
            pl.BlockSpec((1, 1, N2), lambda g, i: (g, 0, 0)),
        ],
        out_shape=[jax.ShapeDtypeStruct((2, N2, DH), jnp.float32),
                   jax.ShapeDtypeStruct((2, N2, DH), jnp.float32),
                   jax.ShapeDtypeStruct((2, 1, N2), jnp.float32),
                   jax.ShapeDtypeStruct((2, 1, N2), jnp.float32)],
    )(x, w2, al2, ar2)


def _sc_edges(fa, fb, el_flat, er_flat, src0, src1, dst0, dst1):
    mesh = plsc.VectorSubcoreMesh(core_axis_name="c", subcore_axis_name="s")

    @functools.partial(
        pl.kernel,
        out_type=[jax.ShapeDtypeStruct((2 * 2 * NC * N2, DH), jnp.float32),
                  jax.ShapeDtypeStruct((2 * NC * DR, D), jnp.float32)],
        mesh=mesh,
        scratch_types=[
            pltpu.VMEM((N,), jnp.float32),
            pltpu.VMEM((N,), jnp.float32),
            pltpu.VMEM((EPT,), jnp.int32),
            pltpu.VMEM((NCHUNK, CH), jnp.int32),
            pltpu.VMEM((EPT,), jnp.float32),
            pltpu.VMEM((DR, D), jnp.float32),
            pltpu.VMEM((DR,), jnp.int32),
            pltpu.VMEM((CH, DH), jnp.float32),
            pltpu.VMEM((CH, DH), jnp.float32),
            pltpu.VMEM((CH, DH), jnp.float32),
            pltpu.VMEM((CH, DH), jnp.float32),
            pltpu.VMEM((ZR, DH), jnp.float32),
            pltpu.VMEM_SHARED((N2, DH), jnp.float32),
            pltpu.VMEM_SHARED((DR, D), jnp.float32),
            pltpu.SemaphoreType.DMA,
            pltpu.SemaphoreType.DMA,
            pltpu.SemaphoreType.DMA,
            pltpu.SemaphoreType.DMA,
            pltpu.SemaphoreType.DMA,
        ],
        compiler_params=_sc_compiler_params(),
    )
    def body(fa_h, fb_h, el_h, er_h, src0_h, src1_h, dst0_h, dst1_h,
             out_h, den_h,
             el_v, er_v, srcb, dstb, sbuf, den_v, rowid,
             grows0, grows1, stage0, stage1,
             zbuf, acc, accd, gsem0, gsem1, ssem0, ssem1, zsem):
        c = lax.axis_index("c")
        s = lax.axis_index("s")
        wid = c * NS + s
        zvec = jnp.zeros((L,), jnp.float32)
        lane = lax.iota(jnp.int32, L)
        lane_masks = [lane == j for j in range(L)]

        @pl.loop(0, ZR)
        def _(r):
            for q in range(DH // L):
                zbuf[r, pl.ds(q * L, L)] = zvec

        for k in range(DR // L):
            rowid[pl.ds(k * L, L)] = lane + k * L

        src_refs = (src0_h, src1_h)
        dst_refs = (dst0_h, dst1_h)
        def fire_zeros():
            for z in range(RPT // ZR):
                pltpu.async_copy(zbuf, acc.at[pl.ds(s * RPT + z * ZR, ZR)],
                                 zsem)

        def drain_zeros():
            for z in range(RPT // ZR):
                pltpu.make_async_copy(zbuf, acc.at[pl.ds(s * RPT, ZR)],
                                      zsem).wait()

        for g in range(2):
            cel = pltpu.async_copy(el_h.at[pl.ds(g * N2, N)], el_v, gsem0)
            cer = pltpu.async_copy(er_h.at[pl.ds(g * N2, N)], er_v, gsem1)
            csrc = pltpu.async_copy(src_refs[g].at[pl.ds(wid * EPT, EPT)],
                                    srcb, ssem0)
            cdst = pltpu.async_copy(
                dst_refs[g].at[pl.ds(wid * NCHUNK, NCHUNK)], dstb, ssem1)
            fire_zeros()

            @pl.loop(0, DR)
            def _(k):
                for q in range(D // L):
                    den_v[k, pl.ds(q * L, L)] = zvec

            @pl.when(s == 0)
            def _():
                pltpu.sync_copy(den_v, accd)

            cel.wait()
            cer.wait()
            csrc.wait()
            cdst.wait()

            @pl.loop(0, NCHUNK)
            def _(r):
                for u in range(CH // L):
                    sl = pl.ds(r * CH + u * L, L)
                    dv = dstb[r, pl.ds(u * L, L)]
                    ev = (plsc.load_gather(el_v, [srcb[sl]])
                          + plsc.load_gather(er_v, [dv]))
                    ev = jnp.where(ev > 0.0, ev, ev * 0.2)
                    sval = jnp.exp(ev)
                    sbuf[sl] = sval
                    drow = lax.shift_right_logical(dv, 7)
                    dcol = jnp.bitwise_and(dv, 127)
                    for j in range(L):
                        plsc.addupdate_scatter(den_v, [drow, dcol], sval,
                                               mask=lane_masks[j])

            def weight(k, grows, stage):
                @plsc.parallel_loop(0, CH // L)
                def _(v):
                    sv = sbuf[pl.ds(k * CH + v * L, L)]
                    for j in range(L):
                        e = v * L + j
                        spl = jnp.broadcast_to(sv[j], (L,))
                        for q in range(DH // L):
                            cs = pl.ds(q * L, L)
                            stage[e, cs] = grows[e, cs] * spl

            for half in range(2):
                tab_g = (fa_h if half == 0 else fb_h).at[pl.ds(g * N2, N2)]
                if half == 1:
                    fire_zeros()
                drain_zeros()
                plsc.subcore_barrier()
                if half == 0:
                    pltpu.sync_copy(den_v, accd.at[rowid], add=True)

                def gather(k, grows, gsem):
                    idx = srcb.at[pl.ds(k * CH, CH)]
                    return pltpu.async_copy(tab_g.at[idx], grows, gsem)

                def gather_wait(grows, gsem):
                    idx = srcb.at[pl.ds(0, CH)]
                    pltpu.make_async_copy(tab_g.at[idx], grows, gsem).wait()

                def scatter(k, stage, ssem):
                    return pltpu.async_copy(stage, acc.at[dstb.at[k]], ssem,
                                            add=True)

                def scatter_wait(stage, ssem):
                    pltpu.make_async_copy(stage, acc.at[dstb.at[0]],
                                          ssem).wait()

                NPAIR = (NCHUNK + 1) // 2
                gather(0, grows0, gsem0)
                gather(1, grows1, gsem1)

                @pl.loop(0, NPAIR)
                def _(i):
                    k0 = 2 * i
                    k1 = k0 + 1

                    @pl.when(i > 0)
                    def _():
                        scatter_wait(stage0, ssem0)
                        scatter_wait(stage1, ssem1)

                    gather_wait(grows0, gsem0)
                    weight(k0, grows0, stage0)
                    scatter(k0, stage0, ssem0)

                    @pl.when(i < NPAIR - 1)
                    def _():
                        gather(k0 + 2, grows0, gsem0)

                    @pl.when(i < NPAIR - 1)
                    def _():
                        gather_wait(grows1, gsem1)
                        weight(k1, grows1, stage1)
                        scatter(k1, stage1, ssem1)

                        @pl.when(i < NPAIR - 2)
                        def _():
                            gather(k1 + 2, grows1, gsem1)

                scatter_wait(stage0, ssem0)

                plsc.subcore_barrier()
                if half == 0:
                    @pl.when(s == 0)
                    def _():
                        pltpu.sync_copy(
                            accd, den_h.at[pl.ds((g * NC + c) * DR, DR)])
                obase = ((g * 2 + half) * NC) * N2
                for z in range(RPT // ZR):
                    rs = s * RPT + z * ZR
                    pltpu.sync_copy(acc.at[pl.ds(rs, ZR)],
                                    out_h.at[pl.ds(obase + c * N2 + rs, ZR)])
                plsc.subcore_barrier()

    return body(fa, fb, el_flat, er_flat, src0, src1, dst0, dst1)


def _tc_combine(partials, dens, swa, sba, swb):
    def body(pa_ref, pb_ref, d_ref, swa_ref, sba_ref, swb_ref, h_ref, pr_ref):
        i = pl.program_id(1)
        pa = pa_ref[0, 0]
        pb = pb_ref[0, 0]
        numer = jnp.concatenate([pa[0] + pa[1], pb[0] + pb[1]], axis=-1)
        den = d_ref[0, 0] + d_ref[0, 1]
        x = numer / (den[:, None] + 1e-9)
        h = jnp.where(x > 0.0, x, jnp.exp(x) - 1.0)
        h_ref[0] = h
        t = jnp.tanh(jnp.dot(h, swa_ref[...],
                             preferred_element_type=jnp.float32)
                     + sba_ref[0][None, :])
        pr = jnp.sum(t * swb_ref[0][None, :], axis=1)
        row = i * BLK + lax.broadcasted_iota(jnp.int32, (BLK,), 0)
        pr = jnp.where(row < N, pr, 0.0)
        pr_ref[0, 0, pl.ds(i * BLK, BLK)] = pr

    return pl.pallas_call(
        body,
        grid=(2, N2 // BLK),
        in_specs=[
            pl.BlockSpec((1, 1, NC, BLK, DH), lambda g, i: (g, 0, 0, i, 0)),
            pl.BlockSpec((1, 1, NC, BLK, DH), lambda g, i: (g, 1, 0, i, 0)),
            pl.BlockSpec((1, NC, BLK), lambda g, i: (g, 0, i)),
            pl.BlockSpec((D, D), lambda g, i: (0, 0)),
            pl.BlockSpec((1, D), lambda g, i: (0, 0)),
            pl.BlockSpec((1, D), lambda g, i: (0, 0)),
        ],
        out_specs=[
            pl.BlockSpec((1, BLK, D), lambda g, i: (g, i, 0)),
            pl.BlockSpec((1, 1, N2), lambda g, i: (g, 0, 0)),
        ],
        out_shape=[jax.ShapeDtypeStruct((2, N2, D), jnp.float32),
                   jax.ShapeDtypeStruct((2, 1, N2), jnp.float32)],
    )(partials, partials, dens, swa, sba, swb)


def _tc_epilogue(h, p, lw, lb, lng1, lnb1):
    def body(ha_ref, hb_ref, p_ref, lw_ref, lb_ref, lng_ref, lnb_ref, o_ref):
        w = jnp.sum(p_ref[...], axis=(1, 2)) / N
        m = jnp.maximum(w[0], w[1])
        e0 = jnp.exp(w[0] - m)
        e1 = jnp.exp(w[1] - m)
        b0 = e0 / (e0 + e1)
        b1 = e1 / (e0 + e1)
        x = b0 * ha_ref[0] + b1 * hb_ref[0]
        y = jnp.dot(x, lw_ref[...], preferred_element_type=jnp.float32) + lb_ref[0][None, :]
        y = jnp.maximum(y, 0.0)
        mu = jnp.mean(y, axis=1, keepdims=True)
        var = jnp.mean((y - mu) ** 2, axis=1, keepdims=True)
        o_ref[...] = ((y - mu) / jnp.sqrt(var + 1e-5)
                      * lng_ref[0][None, :] + lnb_ref[0][None, :])

    return pl.pallas_call(
        body,
        grid=(N2 // BLK,),
        in_specs=[
            pl.BlockSpec((1, BLK, D), lambda i: (0, i, 0)),
            pl.BlockSpec((1, BLK, D), lambda i: (1, i, 0)),
            pl.BlockSpec((2, 1, N2), lambda i: (0, 0, 0)),
            pl.BlockSpec((D, D), lambda i: (0, 0)),
            pl.BlockSpec((1, D), lambda i: (0, 0)),
            pl.BlockSpec((1, D), lambda i: (0, 0)),
            pl.BlockSpec((1, D), lambda i: (0, 0)),
        ],
        out_specs=pl.BlockSpec((BLK, D), lambda i: (i, 0)),
        out_shape=jax.ShapeDtypeStruct((N2, D), jnp.float32),
    )(h, h, p, lw, lb, lng1, lnb1)


def _sc_gather(tab_u, tab_i, idx_u, idx_i):
    mesh = plsc.VectorSubcoreMesh(core_axis_name="c", subcore_axis_name="s")
    BU = idx_u.shape[0]
    BI = idx_i.shape[0]
    upt = BU // NW
    ipt = BI // NW
    GCH = 128

    @functools.partial(
        pl.kernel,
        out_type=jax.ShapeDtypeStruct((BU + BI, D), jnp.float32),
        mesh=mesh,
        scratch_types=[
            pltpu.VMEM((GCH,), jnp.int32),
            pltpu.VMEM((GCH, D), jnp.float32),
            pltpu.SemaphoreType.DMA,
        ],
        compiler_params=_sc_compiler_params(),
    )
    def body(tu_h, ti_h, iu_h, ii_h, out_h, idxb, rows, sem):
        c = lax.axis_index("c")
        s = lax.axis_index("s")
        wid = c * NS + s
        for ck in range(upt // GCH):
            base = wid * upt + ck * GCH
            pltpu.sync_copy(iu_h.at[pl.ds(base, GCH)], idxb)
            pltpu.async_copy(tu_h.at[idxb], rows, sem).wait()
            pltpu.sync_copy(rows, out_h.at[pl.ds(base, GCH)])
        for ck in range(ipt // GCH):
            base = wid * ipt + ck * GCH
            pltpu.sync_copy(ii_h.at[pl.ds(base, GCH)], idxb)
            pltpu.async_copy(ti_h.at[idxb], rows, sem).wait()
            pltpu.sync_copy(rows, out_h.at[pl.ds(BU + base, GCH)])

    return body(tab_u, tab_i, idx_u, idx_i)


def _run_type(x, w0, w1, al0, al1, ar0, ar1, swa, sba, swb, lw, lb,
              lng1, lnb1, src0, src1, dst0, dst1):
    f32 = jnp.float32
    w2 = jnp.stack([w0, w1]).astype(f32)
    al2 = jnp.stack([al0.reshape(-1), al1.reshape(-1)]).astype(f32)
    al2 = al2.reshape(2, 1, D)
    ar2 = jnp.stack([ar0.reshape(-1), ar1.reshape(-1)]).astype(f32)
    ar2 = ar2.reshape(2, 1, D)
    fa, fb, el, er = _tc_prep(x.astype(f32), w2, al2, ar2)
    partials_flat, dens_flat = _sc_edges(
        fa.reshape(2 * N2, DH), fb.reshape(2 * N2, DH),
        el.reshape(2 * N2), er.reshape(2 * N2),
        src0.astype(jnp.int32), src1.astype(jnp.int32),
        dst0.astype(jnp.int32).reshape(NW * NCHUNK, CH),
        dst1.astype(jnp.int32).reshape(NW * NCHUNK, CH))
    partials = partials_flat.reshape(2, 2, NC, N2, DH)
    dens = dens_flat.reshape(2, NC, N2)
    h, p = _tc_combine(partials, dens, swa.astype(f32),
                       sba.astype(f32).reshape(1, D),
                       swb.astype(f32).reshape(1, D))
    return _tc_epilogue(h, p, lw.astype(f32), lb.astype(f32).reshape(1, D),
                        lng1, lnb1)


def kernel(user_idx, item_idx, neg_item_idx, feat_user, feat_item,
           su0, du0, wu0, alu0, aru0, su1, du1, wu1, alu1, aru1,
           si0, di0, wi0, ali0, ari0, si1, di1, wi1, ali1, ari1,
           swa_u, sba_u, swb_u, swa_i, sba_i, swb_i,
           ulw, ulb, ilw, ilb, lng, lnb):
    f32 = jnp.float32
    lng1 = lng.reshape(1, D).astype(f32)
    lnb1 = lnb.reshape(1, D).astype(f32)

    emb_u = _run_type(feat_user, wu0, wu1, alu0, alu1, aru0, aru1,
                      swa_u, sba_u, swb_u, ulw, ulb, lng1, lnb1,
                      su0, su1, du0, du1)
    emb_i = _run_type(feat_item, wi0, wi1, ali0, ali1, ari0, ari1,
                      swa_i, sba_i, swb_i, ilw, ilb, lng1, lnb1,
                      si0, si1, di0, di1)

    idx_u = user_idx.astype(jnp.int32)
    idx_i = jnp.concatenate([item_idx, neg_item_idx]).astype(jnp.int32)
    gathered = _sc_gather(emb_u, emb_i, idx_u, idx_i)
    b = user_idx.shape[0]
    return (gathered[:b], gathered[b:2 * b], gathered[2 * b:])
